# baseline (speedup 1.0000x reference)
import functools

import jax
import jax.numpy as jnp
import numpy as np
from jax.experimental import pallas as pl
from jax.experimental.pallas import tpu as pltpu

D = 4096
H = 2048
E = 64
TOP_K = 8

BM = 512


def _router_kernel(x_ref, w1_ref, b1_ref, w2_ref, b2_ref,
                   probs_ref, idx_ref, rw_ref):
    xb = x_ref[...].astype(jnp.bfloat16)
    acc = jnp.dot(xb, w1_ref[...], preferred_element_type=jnp.float32)
    h = jnp.maximum(acc + b1_ref[...], 0.0)
    logits = jnp.dot(h.astype(jnp.bfloat16), w2_ref[...],
                     preferred_element_type=jnp.float32)
    logits = logits + b2_ref[...]

    lt = logits.T

    m = jnp.max(lt, axis=0, keepdims=True)
    e = jnp.exp(lt - m)
    pt = e / jnp.sum(e, axis=0, keepdims=True)
    probs_ref[...] = pt.T

    rows = jax.lax.broadcasted_iota(jnp.int32, (E, BM), 0)
    work = pt
    vals = []
    idxs = []
    for _ in range(TOP_K):
        mx = jnp.max(work, axis=0, keepdims=True)
        cand = jnp.where(work == mx, rows, E)
        ix = jnp.min(cand, axis=0, keepdims=True)
        vals.append(mx)
        idxs.append(ix)
        work = jnp.where(rows == ix, -1.0, work)
    tkv = jnp.concatenate(vals, axis=0)
    tki = jnp.concatenate(idxs, axis=0)
    idx_ref[...] = tki.T
    m2 = jnp.max(tkv, axis=0, keepdims=True)
    e2 = jnp.exp(tkv - m2)
    rw = e2 / jnp.sum(e2, axis=0, keepdims=True)
    rw_ref[...] = rw.T


def _router_impl(features, W1, b1, W2, b2):
    B, S, _ = features.shape
    M = B * S
    x = features.reshape(M, D)
    b1r = b1.reshape(1, H)
    b2r = b2.reshape(1, E)
    w1b = W1.astype(jnp.bfloat16)
    w2b = W2.astype(jnp.bfloat16)

    grid = (M // BM,)
    probs, idx, rw = pl.pallas_call(
        _router_kernel,
        grid=grid,
        in_specs=[
            pl.BlockSpec((BM, D), lambda m: (m, 0)),
            pl.BlockSpec((D, H), lambda m: (0, 0)),
            pl.BlockSpec((1, H), lambda m: (0, 0)),
            pl.BlockSpec((H, E), lambda m: (0, 0)),
            pl.BlockSpec((1, E), lambda m: (0, 0)),
        ],
        out_specs=[
            pl.BlockSpec((BM, E), lambda m: (m, 0)),
            pl.BlockSpec((BM, TOP_K), lambda m: (m, 0)),
            pl.BlockSpec((BM, TOP_K), lambda m: (m, 0)),
        ],
        out_shape=[
            jax.ShapeDtypeStruct((M, E), jnp.float32),
            jax.ShapeDtypeStruct((M, TOP_K), jnp.int32),
            jax.ShapeDtypeStruct((M, TOP_K), jnp.float32),
        ],
        compiler_params=pltpu.CompilerParams(
            dimension_semantics=("arbitrary",),
        ),
    )(x, w1b, b1r, w2b, b2r)

    return (probs.reshape(B, S, E),
            idx.reshape(B, S, TOP_K),
            rw.reshape(B, S, TOP_K))


@functools.partial(jax.jit, static_argnames=())
def kernel(features, W1, b1, W2, b2):
    devs = jax.devices()
    nd = len(devs)
    if nd > 1 and features.shape[0] % nd == 0:
        mesh = jax.sharding.Mesh(np.asarray(devs), ("d",))
        P = jax.sharding.PartitionSpec
        f = jax.shard_map(
            _router_impl, mesh=mesh,
            in_specs=(P("d"), P(), P(), P(), P()),
            out_specs=(P("d"), P("d"), P("d")),
            check_vma=False)
        return f(features, W1, b1, W2, b2)
    return _router_impl(features, W1, b1, W2, b2)


# speedup vs baseline: 1.9574x; 1.9574x over previous
import functools

import jax
import jax.numpy as jnp
import numpy as np
from jax.experimental import pallas as pl
from jax.experimental.pallas import tpu as pltpu

D = 4096
H = 2048
E = 64
TOP_K = 8

BM = 512


def _router_kernel(x_ref, w1_ref, b1_ref, w2_ref, b2_ref,
                   probs_ref, idx_ref, rw_ref):
    xb = x_ref[...].astype(jnp.bfloat16)
    acc = jnp.dot(xb, w1_ref[...], preferred_element_type=jnp.float32)
    h = jnp.maximum(acc + b1_ref[...], 0.0)
    logits = jnp.dot(h.astype(jnp.bfloat16), w2_ref[...],
                     preferred_element_type=jnp.float32)
    logits = logits + b2_ref[...]

    lt = logits.T

    m = jnp.max(lt, axis=0, keepdims=True)
    e = jnp.exp(lt - m)
    pt = e / jnp.sum(e, axis=0, keepdims=True)
    probs_ref[...] = pt.T

    rows = jax.lax.broadcasted_iota(jnp.int32, (E, BM), 0)
    work = pt
    vals = []
    idxs = []
    for _ in range(TOP_K):
        mx = jnp.max(work, axis=0, keepdims=True)
        cand = jnp.where(work == mx, rows, E)
        ix = jnp.min(cand, axis=0, keepdims=True)
        vals.append(mx)
        idxs.append(ix)
        work = jnp.where(rows == ix, -1.0, work)
    tkv = jnp.concatenate(vals, axis=0)
    tki = jnp.concatenate(idxs, axis=0)
    idx_ref[...] = tki.T
    m2 = jnp.max(tkv, axis=0, keepdims=True)
    e2 = jnp.exp(tkv - m2)
    rw = e2 / jnp.sum(e2, axis=0, keepdims=True)
    rw_ref[...] = rw.T


def _router_impl(features, W1, b1, W2, b2):
    B, S, _ = features.shape
    M = B * S
    x = features.reshape(M, D)
    b1r = b1.reshape(1, H)
    b2r = b2.reshape(1, E)
    w1b = W1.astype(jnp.bfloat16)
    w2b = W2.astype(jnp.bfloat16)

    grid = (M // BM,)
    probs, idx, rw = pl.pallas_call(
        _router_kernel,
        grid=grid,
        in_specs=[
            pl.BlockSpec((BM, D), lambda m: (m, 0)),
            pl.BlockSpec((D, H), lambda m: (0, 0)),
            pl.BlockSpec((1, H), lambda m: (0, 0)),
            pl.BlockSpec((H, E), lambda m: (0, 0)),
            pl.BlockSpec((1, E), lambda m: (0, 0)),
        ],
        out_specs=[
            pl.BlockSpec((BM, E), lambda m: (m, 0)),
            pl.BlockSpec((BM, TOP_K), lambda m: (m, 0)),
            pl.BlockSpec((BM, TOP_K), lambda m: (m, 0)),
        ],
        out_shape=[
            jax.ShapeDtypeStruct((M, E), jnp.float32),
            jax.ShapeDtypeStruct((M, TOP_K), jnp.int32),
            jax.ShapeDtypeStruct((M, TOP_K), jnp.float32),
        ],
        compiler_params=pltpu.CompilerParams(
            dimension_semantics=("arbitrary",),
        ),
    )(x, w1b, b1r, w2b, b2r)

    return (probs.reshape(B, S, E),
            idx.reshape(B, S, TOP_K),
            rw.reshape(B, S, TOP_K))


@functools.partial(jax.jit, static_argnames=())
def kernel(features, W1, b1, W2, b2):
    return _router_impl(features, W1, b1, W2, b2)


# speedup vs baseline: 1.9866x; 1.0149x over previous
import functools

import jax
import jax.numpy as jnp
import numpy as np
from jax.experimental import pallas as pl
from jax.experimental.pallas import tpu as pltpu

D = 4096
H = 2048
E = 64
TOP_K = 8

BM = 512
HB = BM // 2


def _epilogue(logits, n):
    lt = logits.T
    m = jnp.max(lt, axis=0, keepdims=True)
    e = jnp.exp(lt - m)
    pt = e / jnp.sum(e, axis=0, keepdims=True)

    rows = jax.lax.broadcasted_iota(jnp.int32, (E, n), 0)
    work = pt
    vals = []
    idxs = []
    for _ in range(TOP_K):
        mx = jnp.max(work, axis=0, keepdims=True)
        cand = jnp.where(work == mx, rows, E)
        ix = jnp.min(cand, axis=0, keepdims=True)
        vals.append(mx)
        idxs.append(ix)
        work = jnp.where(rows == ix, -1.0, work)
    tkv = jnp.concatenate(vals, axis=0)
    tki = jnp.concatenate(idxs, axis=0)
    m2 = jnp.max(tkv, axis=0, keepdims=True)
    e2 = jnp.exp(tkv - m2)
    rw = e2 / jnp.sum(e2, axis=0, keepdims=True)
    return pt.T, tki.T, rw.T


def _router_kernel(x_ref, w1_ref, b1_ref, w2_ref, b2_ref,
                   probs_ref, idx_ref, rw_ref):
    w1 = w1_ref[...]
    w2 = w2_ref[...]
    b1v = b1_ref[...]
    b2v = b2_ref[...]

    def logits_half(lo):
        xb = x_ref[pl.ds(lo, HB), :].astype(jnp.bfloat16)
        acc = jnp.dot(xb, w1, preferred_element_type=jnp.float32)
        h = jnp.maximum(acc + b1v, 0.0)
        lg = jnp.dot(h.astype(jnp.bfloat16), w2,
                     preferred_element_type=jnp.float32)
        return lg + b2v

    lg0 = logits_half(0)
    lg1 = logits_half(HB)
    p0, i0, r0 = _epilogue(lg0, HB)
    p1, i1, r1 = _epilogue(lg1, HB)
    probs_ref[pl.ds(0, HB), :] = p0
    idx_ref[pl.ds(0, HB), :] = i0
    rw_ref[pl.ds(0, HB), :] = r0
    probs_ref[pl.ds(HB, HB), :] = p1
    idx_ref[pl.ds(HB, HB), :] = i1
    rw_ref[pl.ds(HB, HB), :] = r1


def _router_impl(features, W1, b1, W2, b2):
    B, S, _ = features.shape
    M = B * S
    x = features.reshape(M, D)
    b1r = b1.reshape(1, H)
    b2r = b2.reshape(1, E)
    w1b = W1.astype(jnp.bfloat16)
    w2b = W2.astype(jnp.bfloat16)

    grid = (M // BM,)
    probs, idx, rw = pl.pallas_call(
        _router_kernel,
        grid=grid,
        in_specs=[
            pl.BlockSpec((BM, D), lambda m: (m, 0)),
            pl.BlockSpec((D, H), lambda m: (0, 0)),
            pl.BlockSpec((1, H), lambda m: (0, 0)),
            pl.BlockSpec((H, E), lambda m: (0, 0)),
            pl.BlockSpec((1, E), lambda m: (0, 0)),
        ],
        out_specs=[
            pl.BlockSpec((BM, E), lambda m: (m, 0)),
            pl.BlockSpec((BM, TOP_K), lambda m: (m, 0)),
            pl.BlockSpec((BM, TOP_K), lambda m: (m, 0)),
        ],
        out_shape=[
            jax.ShapeDtypeStruct((M, E), jnp.float32),
            jax.ShapeDtypeStruct((M, TOP_K), jnp.int32),
            jax.ShapeDtypeStruct((M, TOP_K), jnp.float32),
        ],
        compiler_params=pltpu.CompilerParams(
            dimension_semantics=("arbitrary",),
        ),
    )(x, w1b, b1r, w2b, b2r)

    return (probs.reshape(B, S, E),
            idx.reshape(B, S, TOP_K),
            rw.reshape(B, S, TOP_K))


@functools.partial(jax.jit, static_argnames=())
def kernel(features, W1, b1, W2, b2):
    return _router_impl(features, W1, b1, W2, b2)
